# baseline (speedup 1.0000x reference)
import jax
import jax.numpy as jnp
from jax.experimental import pallas as pl
from jax.experimental.pallas import tpu as pltpu

_EXPERTS = 64
_BLOCK_T = 8192
_KSPLIT = 2
_CHUNK_T = 1024


def _gate_block(x_ref, w_ref, y_ref, logits_ref, acc_ref):
    k = pl.program_id(1)
    part = jax.lax.dot_general(
        x_ref[...], w_ref[...], (((1,), (1,)), ((), ())),
        preferred_element_type=jnp.float32,
    )

    @pl.when(k == 0)
    def _init():
        acc_ref[...] = part

    @pl.when(k == _KSPLIT - 1)
    def _epilogue():
        acc_ref[...] += part

        def _chunk(c, _):
            sl = pl.ds(c * _CHUNK_T, _CHUNK_T)
            logits = acc_ref[sl, :]
            logits_ref[sl, :] = logits
            m = jnp.max(logits, axis=1, keepdims=True)
            e = jnp.exp(logits - m)
            s = jnp.sum(e, axis=1, keepdims=True)
            col = jax.lax.broadcasted_iota(
                jnp.int32, logits.shape, 1).astype(jnp.float32)
            i1 = jnp.min(jnp.where(logits == m, col, jnp.float32(_EXPERTS)),
                         axis=1, keepdims=True)
            at1 = col == i1
            l2 = jnp.where(at1, jnp.float32(-jnp.inf), logits)
            m2 = jnp.max(l2, axis=1, keepdims=True)
            keep = at1 | (l2 == m2)
            y_ref[sl, :] = jnp.where(keep, e / s, jnp.float32(0.0))
            return ()

        jax.lax.fori_loop(0, _BLOCK_T // _CHUNK_T, _chunk, ())


def kernel(x, W):
    n_tokens, k_dim = x.shape
    bk = k_dim // _KSPLIT
    grid = (n_tokens // _BLOCK_T, _KSPLIT)
    y, logits = pl.pallas_call(
        _gate_block,
        grid=grid,
        in_specs=[
            pl.BlockSpec((_BLOCK_T, bk), lambda i, k: (i, k)),
            pl.BlockSpec((W.shape[0], bk), lambda i, k: (0, k)),
        ],
        out_specs=[
            pl.BlockSpec((_BLOCK_T, _EXPERTS), lambda i, k: (i, 0)),
            pl.BlockSpec((_BLOCK_T, _EXPERTS), lambda i, k: (i, 0)),
        ],
        out_shape=[
            jax.ShapeDtypeStruct((n_tokens, _EXPERTS), jnp.float32),
            jax.ShapeDtypeStruct((n_tokens, _EXPERTS), jnp.float32),
        ],
        scratch_shapes=[pltpu.VMEM((_BLOCK_T, _EXPERTS), jnp.float32)],
    )(x, W)
    return (y, logits)


# speedup vs baseline: 1.2977x; 1.2977x over previous
import jax
import jax.numpy as jnp
from jax.experimental import pallas as pl

_EXPERTS = 64
_BLOCK_T = 4096


def _gate_block(x_ref, w_ref, y_ref, logits_ref):
    logits = jax.lax.dot_general(
        x_ref[...], w_ref[...], (((1,), (1,)), ((), ())),
        preferred_element_type=jnp.float32,
    )
    logits_ref[...] = logits
    m = jnp.max(logits, axis=1, keepdims=True)
    e = jnp.exp(logits - m)
    s = jnp.sum(e, axis=1, keepdims=True)
    col = jax.lax.broadcasted_iota(jnp.int32, logits.shape, 1).astype(
        jnp.float32)
    i1 = jnp.min(jnp.where(logits == m, col, jnp.float32(_EXPERTS)),
                 axis=1, keepdims=True)
    at1 = col == i1
    l2 = jnp.where(at1, jnp.float32(-jnp.inf), logits)
    m2 = jnp.max(l2, axis=1, keepdims=True)
    keep = at1 | (l2 == m2)
    y_ref[...] = jnp.where(keep, e / s, jnp.float32(0.0))


def kernel(x, W):
    n_tokens, k_dim = x.shape
    grid = (n_tokens // _BLOCK_T,)
    y, logits = pl.pallas_call(
        _gate_block,
        grid=grid,
        in_specs=[
            pl.BlockSpec((_BLOCK_T, k_dim), lambda i: (i, 0)),
            pl.BlockSpec(W.shape, lambda i: (0, 0)),
        ],
        out_specs=[
            pl.BlockSpec((_BLOCK_T, _EXPERTS), lambda i: (i, 0)),
            pl.BlockSpec((_BLOCK_T, _EXPERTS), lambda i: (i, 0)),
        ],
        out_shape=[
            jax.ShapeDtypeStruct((n_tokens, _EXPERTS), jnp.float32),
            jax.ShapeDtypeStruct((n_tokens, _EXPERTS), jnp.float32),
        ],
    )(x, W)
    return (y, logits)
